# baseline (speedup 1.0000x reference)
import jax
import jax.numpy as jnp
from jax import lax
from jax.experimental import pallas as pl
from jax.experimental.pallas import tpu as pltpu


def _round_up(x: int, m: int) -> int:
    return ((x + m - 1) // m) * m


def _make_esn_body(alpha: float, chunk: int, tail: int, num_chunks: int,
                   unroll: int):
    alpha = float(alpha)

    def _body(x_ref, r0_ref, at_ref, bt_ref, bias_ref, r_ref, u_scr):
        c = pl.program_id(1)

        @pl.when(c == 0)
        def _():
            r_ref[...] = r0_ref[...]

        a_t = at_ref[...]
        b_t = bt_ref[...]
        bias_row = bias_ref[...]

        def ustep(t, _):
            u_scr[t] = jnp.dot(x_ref[t], b_t,
                               preferred_element_type=jnp.float32) + bias_row
            return 0

        def step(t, r):
            pre = jnp.dot(r, a_t, preferred_element_type=jnp.float32) + u_scr[t]
            return r + alpha * (jnp.tanh(pre) - r)

        if tail == chunk:
            lax.fori_loop(0, chunk, ustep, 0, unroll=min(unroll, chunk))
            r_ref[...] = lax.fori_loop(0, chunk, step, r_ref[...],
                                       unroll=min(unroll, chunk))
        else:
            @pl.when(c < num_chunks - 1)
            def _():
                lax.fori_loop(0, chunk, ustep, 0, unroll=min(unroll, chunk))
                r_ref[...] = lax.fori_loop(0, chunk, step, r_ref[...],
                                           unroll=min(unroll, chunk))

            @pl.when(c == num_chunks - 1)
            def _():
                lax.fori_loop(0, tail, ustep, 0, unroll=min(unroll, tail))
                r_ref[...] = lax.fori_loop(0, tail, step, r_ref[...],
                                           unroll=min(unroll, tail))

    return _body


def kernel(inp, r_0, A, B, bias, C):
    alpha = 0.6
    batch, seq_len, nin = inp.shape
    nr = A.shape[0]

    nr_p = _round_up(nr, 128)
    batch_8 = _round_up(batch, 8)
    tb = min(batch_8, 128)
    batch_p = _round_up(batch_8, tb)
    nb = batch_p // tb
    f_p = _round_up(nin, 8)

    budget = (52 << 20) - 2 * nr_p * nr_p * 4 - (2 << 20)
    per_step = tb * nr_p * 4 + 2 * tb * 128 * 4
    max_chunk = int(max(8, min(256, seq_len, budget // per_step)))
    chunk = max_chunk
    for c in range(max_chunk, 7, -1):
        if seq_len % c == 0:
            chunk = c
            break
    num_chunks = int(pl.cdiv(seq_len, chunk))
    seq_p = num_chunks * chunk
    tail = seq_len - (num_chunks - 1) * chunk

    x = jnp.asarray(inp, jnp.float32)
    a_f = jnp.asarray(A, jnp.float32)
    b_f = jnp.asarray(B, jnp.float32)
    bias_f = jnp.asarray(bias, jnp.float32)
    c_f = jnp.asarray(C, jnp.float32)
    r0 = jnp.asarray(r_0, jnp.float32)[0]

    x_tm = jnp.transpose(x, (1, 0, 2))
    x_p = jnp.pad(x_tm, ((0, seq_p - seq_len), (0, batch_p - batch),
                         (0, f_p - nin)))
    r0_p = jnp.pad(r0, ((0, batch_p - batch), (0, nr_p - nr)))
    at_p = jnp.pad(a_f.T, ((0, nr_p - nr), (0, nr_p - nr)))
    bt_p = jnp.pad(b_f.T, ((0, f_p - nin), (0, nr_p - nr)))
    bias_p = jnp.pad(bias_f[None, :], ((0, 0), (0, nr_p - nr)))

    body = _make_esn_body(alpha, chunk, tail, num_chunks, unroll=8)

    vmem_limit = int(58 << 20)

    r_final = pl.pallas_call(
        body,
        out_shape=jax.ShapeDtypeStruct((batch_p, nr_p), jnp.float32),
        grid=(nb, num_chunks),
        in_specs=[
            pl.BlockSpec((chunk, tb, f_p), lambda b, c: (c, b, 0)),
            pl.BlockSpec((tb, nr_p), lambda b, c: (b, 0)),
            pl.BlockSpec((nr_p, nr_p), lambda b, c: (0, 0)),
            pl.BlockSpec((f_p, nr_p), lambda b, c: (0, 0)),
            pl.BlockSpec((1, nr_p), lambda b, c: (0, 0)),
        ],
        out_specs=pl.BlockSpec((tb, nr_p), lambda b, c: (b, 0)),
        scratch_shapes=[pltpu.VMEM((chunk, tb, nr_p), jnp.float32)],
        compiler_params=pltpu.CompilerParams(
            dimension_semantics=("parallel", "arbitrary"),
            vmem_limit_bytes=vmem_limit,
        ),
    )(x_p, r0_p, at_p, bt_p, bias_p)

    return r_final[:batch, :nr] @ c_f.T


# speedup vs baseline: 1.0453x; 1.0453x over previous
import jax
import jax.numpy as jnp
from jax import lax
from jax.experimental import pallas as pl
from jax.experimental.pallas import tpu as pltpu


def _round_up(x: int, m: int) -> int:
    return ((x + m - 1) // m) * m


def _make_esn_body(alpha: float, chunk: int, tail: int, num_chunks: int,
                   unroll: int):
    alpha = float(alpha)

    def _body(x_ref, r0_ref, at_ref, bt_ref, bias_ref, r_ref, u_scr):
        c = pl.program_id(1)

        @pl.when(c == 0)
        def _():
            r_ref[...] = r0_ref[...]

        a_t = at_ref[...]
        b_t = bt_ref[...]
        bias_row = bias_ref[...]

        def ustep(t, _):
            u_scr[t] = jnp.dot(x_ref[t], b_t,
                               preferred_element_type=jnp.float32) + bias_row
            return 0

        def step(t, r):
            pre = jnp.dot(r, a_t, preferred_element_type=jnp.float32) + u_scr[t]
            return r + alpha * (jnp.tanh(pre) - r)

        if tail == chunk:
            lax.fori_loop(0, chunk, ustep, 0, unroll=min(unroll, chunk))
            r_ref[...] = lax.fori_loop(0, chunk, step, r_ref[...],
                                       unroll=min(unroll, chunk))
        else:
            @pl.when(c < num_chunks - 1)
            def _():
                lax.fori_loop(0, chunk, ustep, 0, unroll=min(unroll, chunk))
                r_ref[...] = lax.fori_loop(0, chunk, step, r_ref[...],
                                           unroll=min(unroll, chunk))

            @pl.when(c == num_chunks - 1)
            def _():
                lax.fori_loop(0, tail, ustep, 0, unroll=min(unroll, tail))
                r_ref[...] = lax.fori_loop(0, tail, step, r_ref[...],
                                           unroll=min(unroll, tail))

    return _body


def kernel(inp, r_0, A, B, bias, C):
    alpha = 0.6
    batch, seq_len, nin = inp.shape
    nr = A.shape[0]

    nr_p = _round_up(nr, 128)
    batch_8 = _round_up(batch, 8)
    tb = min(batch_8, 128)
    batch_p = _round_up(batch_8, tb)
    nb = batch_p // tb
    f_p = _round_up(nin, 8)

    budget = (46 << 20) - 2 * nr_p * nr_p * 4 - (2 << 20)
    per_step = tb * nr_p * 4 + 2 * tb * 128 * 4
    max_chunk = int(max(8, min(256, seq_len, budget // per_step)))
    chunk = max_chunk
    for c in range(max_chunk, 7, -1):
        if seq_len % c == 0:
            chunk = c
            break
    num_chunks = int(pl.cdiv(seq_len, chunk))
    seq_p = num_chunks * chunk
    tail = seq_len - (num_chunks - 1) * chunk

    x = jnp.asarray(inp, jnp.float32)
    a_f = jnp.asarray(A, jnp.float32)
    b_f = jnp.asarray(B, jnp.float32)
    bias_f = jnp.asarray(bias, jnp.float32)
    c_f = jnp.asarray(C, jnp.float32)
    r0 = jnp.asarray(r_0, jnp.float32)[0]

    x_tm = jnp.transpose(x, (1, 0, 2))
    x_p = jnp.pad(x_tm, ((0, seq_p - seq_len), (0, batch_p - batch),
                         (0, f_p - nin)))
    r0_p = jnp.pad(r0, ((0, batch_p - batch), (0, nr_p - nr)))
    at_p = jnp.pad(a_f.T, ((0, nr_p - nr), (0, nr_p - nr)))
    bt_p = jnp.pad(b_f.T, ((0, f_p - nin), (0, nr_p - nr)))
    bias_p = jnp.pad(bias_f[None, :], ((0, 0), (0, nr_p - nr)))

    body = _make_esn_body(alpha, chunk, tail, num_chunks, unroll=16)

    vmem_limit = int(58 << 20)

    r_final = pl.pallas_call(
        body,
        out_shape=jax.ShapeDtypeStruct((batch_p, nr_p), jnp.float32),
        grid=(nb, num_chunks),
        in_specs=[
            pl.BlockSpec((chunk, tb, f_p), lambda b, c: (c, b, 0)),
            pl.BlockSpec((tb, nr_p), lambda b, c: (b, 0)),
            pl.BlockSpec((nr_p, nr_p), lambda b, c: (0, 0)),
            pl.BlockSpec((f_p, nr_p), lambda b, c: (0, 0)),
            pl.BlockSpec((1, nr_p), lambda b, c: (0, 0)),
        ],
        out_specs=pl.BlockSpec((tb, nr_p), lambda b, c: (b, 0)),
        scratch_shapes=[pltpu.VMEM((chunk, tb, nr_p), jnp.float32)],
        compiler_params=pltpu.CompilerParams(
            dimension_semantics=("parallel", "arbitrary"),
            vmem_limit_bytes=vmem_limit,
        ),
    )(x_p, r0_p, at_p, bt_p, bias_p)

    return r_final[:batch, :nr] @ c_f.T


# speedup vs baseline: 1.0580x; 1.0121x over previous
import jax
import jax.numpy as jnp
from jax import lax
from jax.experimental import pallas as pl
from jax.experimental.pallas import tpu as pltpu


def _round_up(x: int, m: int) -> int:
    return ((x + m - 1) // m) * m


def _make_esn_body(alpha: float, chunk: int, tail: int, num_chunks: int,
                   unroll: int):
    alpha = float(alpha)

    def _body(x_ref, r0_ref, at_ref, bt_ref, bias_ref, r_ref, u_scr):
        c = pl.program_id(1)

        @pl.when(c == 0)
        def _():
            r_ref[...] = r0_ref[...]

        a_t = at_ref[...]
        b_t = bt_ref[...]
        bias_row = bias_ref[...]

        tb, f_p = x_ref.shape[1], x_ref.shape[2]
        nr_p = u_scr.shape[2]

        def upre(n):
            xm = x_ref[0:n].reshape(n * tb, f_p)
            u = jnp.dot(xm, b_t, preferred_element_type=jnp.float32) + bias_row
            u_scr[0:n] = u.reshape(n, tb, nr_p)

        def step(t, r):
            pre = jnp.dot(r, a_t, preferred_element_type=jnp.float32) + u_scr[t]
            return r + alpha * (jnp.tanh(pre) - r)

        if tail == chunk:
            upre(chunk)
            r_ref[...] = lax.fori_loop(0, chunk, step, r_ref[...],
                                       unroll=min(unroll, chunk))
        else:
            @pl.when(c < num_chunks - 1)
            def _():
                upre(chunk)
                r_ref[...] = lax.fori_loop(0, chunk, step, r_ref[...],
                                           unroll=min(unroll, chunk))

            @pl.when(c == num_chunks - 1)
            def _():
                upre(tail)
                r_ref[...] = lax.fori_loop(0, tail, step, r_ref[...],
                                           unroll=min(unroll, tail))

    return _body


def kernel(inp, r_0, A, B, bias, C):
    alpha = 0.6
    batch, seq_len, nin = inp.shape
    nr = A.shape[0]

    nr_p = _round_up(nr, 128)
    batch_8 = _round_up(batch, 8)
    tb = min(batch_8, 128)
    batch_p = _round_up(batch_8, tb)
    nb = batch_p // tb
    f_p = _round_up(nin, 8)

    budget = (46 << 20) - 2 * nr_p * nr_p * 4 - (2 << 20)
    per_step = tb * nr_p * 4 + 2 * tb * 128 * 4
    max_chunk = int(max(8, min(256, seq_len, budget // per_step)))
    chunk = max_chunk
    for c in range(max_chunk, 7, -1):
        if seq_len % c == 0:
            chunk = c
            break
    num_chunks = int(pl.cdiv(seq_len, chunk))
    seq_p = num_chunks * chunk
    tail = seq_len - (num_chunks - 1) * chunk

    x = jnp.asarray(inp, jnp.float32)
    a_f = jnp.asarray(A, jnp.float32)
    b_f = jnp.asarray(B, jnp.float32)
    bias_f = jnp.asarray(bias, jnp.float32)
    c_f = jnp.asarray(C, jnp.float32)
    r0 = jnp.asarray(r_0, jnp.float32)[0]

    x_tm = jnp.transpose(x, (1, 0, 2))
    x_p = jnp.pad(x_tm, ((0, seq_p - seq_len), (0, batch_p - batch),
                         (0, f_p - nin)))
    r0_p = jnp.pad(r0, ((0, batch_p - batch), (0, nr_p - nr)))
    at_p = jnp.pad(a_f.T, ((0, nr_p - nr), (0, nr_p - nr)))
    bt_p = jnp.pad(b_f.T, ((0, f_p - nin), (0, nr_p - nr)))
    bias_p = jnp.pad(bias_f[None, :], ((0, 0), (0, nr_p - nr)))

    body = _make_esn_body(alpha, chunk, tail, num_chunks, unroll=16)

    vmem_limit = int(58 << 20)

    r_final = pl.pallas_call(
        body,
        out_shape=jax.ShapeDtypeStruct((batch_p, nr_p), jnp.float32),
        grid=(nb, num_chunks),
        in_specs=[
            pl.BlockSpec((chunk, tb, f_p), lambda b, c: (c, b, 0)),
            pl.BlockSpec((tb, nr_p), lambda b, c: (b, 0)),
            pl.BlockSpec((nr_p, nr_p), lambda b, c: (0, 0)),
            pl.BlockSpec((f_p, nr_p), lambda b, c: (0, 0)),
            pl.BlockSpec((1, nr_p), lambda b, c: (0, 0)),
        ],
        out_specs=pl.BlockSpec((tb, nr_p), lambda b, c: (b, 0)),
        scratch_shapes=[pltpu.VMEM((chunk, tb, nr_p), jnp.float32)],
        compiler_params=pltpu.CompilerParams(
            dimension_semantics=("parallel", "arbitrary"),
            vmem_limit_bytes=vmem_limit,
        ),
    )(x_p, r0_p, at_p, bt_p, bias_p)

    return r_final[:batch, :nr] @ c_f.T
